# baseline (speedup 1.0000x reference)
import functools

import jax
import jax.numpy as jnp
from jax import lax
from jax.experimental import pallas as pl
from jax.experimental.pallas import tpu as pltpu
from jax.experimental.pallas import tpu_sc as plsc

N = 10000
D_H = 128
D_OUT = 64

NC = 2
NS = 16
NW = NC * NS
CHUNK = 128
N_ACC = 10240
ZROWS = N_ACC // NS

_MESH = plsc.VectorSubcoreMesh(
    core_axis_name="c", subcore_axis_name="s", num_cores=NC, num_subcores=NS)


def _zero_f32(ref, rows, width):
    zv = jnp.zeros((16,), jnp.float32)

    @pl.loop(0, rows)
    def _row(r):
        for k in range(width // 16):
            ref[r, pl.ds(k * 16, 16)] = zv


def _make_degree_kernel(K):
    @functools.partial(
        pl.kernel,
        out_type=jax.ShapeDtypeStruct((NC, 2, N_ACC), jnp.float32),
        mesh=_MESH,
        scratch_types=[
            pltpu.VMEM((K, CHUNK), jnp.int32),
            pltpu.VMEM((1, CHUNK), jnp.float32),
            pltpu.VMEM((1, ZROWS), jnp.float32),
            pltpu.VMEM_SHARED((N_ACC,), jnp.float32),
            pltpu.VMEM_SHARED((N_ACC,), jnp.float32),
        ],
    )
    def deg_kernel(src_hbm, dst_hbm, deg_hbm, idx_v, ones_v, zbuf_v, acc_out, acc_in):
        c = lax.axis_index("c")
        s = lax.axis_index("s")
        wid = c * NS + s

        _zero_f32(zbuf_v, 1, ZROWS)
        for k in range(CHUNK // 16):
            ones_v[0, pl.ds(k * 16, 16)] = jnp.ones((16,), jnp.float32)
        pltpu.sync_copy(zbuf_v.at[0], acc_out.at[pl.ds(s * ZROWS, ZROWS)])
        pltpu.sync_copy(zbuf_v.at[0], acc_in.at[pl.ds(s * ZROWS, ZROWS)])
        plsc.subcore_barrier()

        pltpu.sync_copy(src_hbm.at[wid], idx_v)

        @pl.loop(0, K)
        def _src_chunk(j):
            pltpu.sync_copy(ones_v.at[0], acc_out.at[idx_v.at[j]], add=True)

        pltpu.sync_copy(dst_hbm.at[wid], idx_v)

        @pl.loop(0, K)
        def _dst_chunk(j):
            pltpu.sync_copy(ones_v.at[0], acc_in.at[idx_v.at[j]], add=True)

        plsc.subcore_barrier()
        pltpu.sync_copy(acc_out.at[pl.ds(s * ZROWS, ZROWS)],
                        deg_hbm.at[c, 0, pl.ds(s * ZROWS, ZROWS)])
        pltpu.sync_copy(acc_in.at[pl.ds(s * ZROWS, ZROWS)],
                        deg_hbm.at[c, 1, pl.ds(s * ZROWS, ZROWS)])

    return deg_kernel


_G = 2


def _make_edge_kernel(K, nsrc, pipelined=True):
    D = D_OUT
    assert K % (2 * _G) == 0
    NG = K // _G

    @functools.partial(
        pl.kernel,
        out_type=jax.ShapeDtypeStruct((nsrc, NC, N_ACC, D), jnp.float32),
        mesh=_MESH,
        scratch_types=[
            pltpu.VMEM((K, CHUNK), jnp.int32),
            pltpu.VMEM((K, CHUNK), jnp.int32),
            pltpu.VMEM((2 * _G, CHUNK, D), jnp.float32),
            pltpu.VMEM((128, D), jnp.float32),
            pltpu.VMEM_SHARED((N_ACC, D), jnp.float32),
            pltpu.SemaphoreType.DMA,
            pltpu.SemaphoreType.DMA,
        ],
        compiler_params=pltpu.CompilerParams(use_tc_tiling_on_sc=False),
    )
    def edge_kernel(*refs):
        ys = refs[:nsrc]
        (src_hbm, dst_hbm, out_hbm, idx_s, idx_d, gbuf, zbuf, acc,
         sg0, sg1) = refs[nsrc:]
        sg = (sg0, sg1)
        c = lax.axis_index("c")
        s = lax.axis_index("s")
        wid = c * NS + s

        _zero_f32(zbuf, 128, D)
        pltpu.sync_copy(src_hbm.at[wid], idx_s)
        pltpu.sync_copy(dst_hbm.at[wid], idx_d)

        for p in range(nsrc):
            y_hbm = ys[p]

            def fire_g(st, i):
                for b in range(_G):
                    pltpu.async_copy(y_hbm.at[idx_s.at[i * _G + b]],
                                     gbuf.at[st * _G + b], sg[st])

            def drain_g(st, i):
                for b in range(_G):
                    pltpu.make_async_copy(y_hbm.at[idx_s.at[i * _G + b]],
                                          gbuf.at[st * _G + b], sg[st]).wait()

            def scatter(st, i):
                for b in range(_G):
                    pltpu.sync_copy(gbuf.at[st * _G + b],
                                    acc.at[idx_d.at[i * _G + b]], add=True)

            for b in range(ZROWS // 128):
                pltpu.sync_copy(zbuf, acc.at[pl.ds(s * ZROWS + b * 128, 128)])
            plsc.subcore_barrier()

            if not pipelined:
                @pl.loop(0, K)
                def _chunk(j):
                    pltpu.sync_copy(y_hbm.at[idx_s.at[j]], gbuf.at[0])
                    pltpu.sync_copy(gbuf.at[0], acc.at[idx_d.at[j]], add=True)

                plsc.subcore_barrier()
                pltpu.sync_copy(acc.at[pl.ds(s * ZROWS, ZROWS)],
                                out_hbm.at[p, c, pl.ds(s * ZROWS, ZROWS)])
                continue

            fire_g(0, 0)
            fire_g(1, 1)

            @pl.loop(0, NG, step=2)
            def _pair(i):
                drain_g(0, i)
                scatter(0, i)

                @pl.when(i + 2 < NG)
                def _():
                    fire_g(0, i + 2)

                drain_g(1, i + 1)
                scatter(1, i + 1)

                @pl.when(i + 3 < NG)
                def _():
                    fire_g(1, i + 3)

            plsc.subcore_barrier()
            pltpu.sync_copy(acc.at[pl.ds(s * ZROWS, ZROWS)],
                            out_hbm.at[p, c, pl.ds(s * ZROWS, ZROWS)])

    return edge_kernel


def _norms(deg_ref):
    deg_out = deg_ref[0, 0, :N] + deg_ref[1, 0, :N]
    deg_in = deg_ref[0, 1, :N] + deg_ref[1, 1, :N]
    norm_out = jnp.where(deg_out > 0, lax.rsqrt(jnp.maximum(deg_out, 1.0)), 0.0)
    norm_in = jnp.where(deg_in > 0, lax.rsqrt(jnp.maximum(deg_in, 1.0)), 0.0)
    return norm_out, norm_in


def _mm1_body(deg_ref, x_ref, w_ref, ya_ref, yb_ref):
    norm_out, _ = _norms(deg_ref)
    y = jnp.dot(x_ref[...] * norm_out[:, None], w_ref[...],
                preferred_element_type=jnp.float32)
    ya_ref[...] = y[:, :D_OUT]
    yb_ref[...] = y[:, D_OUT:]


def _mm2_body(deg_ref, p_ref, b1_ref, w_ref, y_ref):
    norm_out, norm_in = _norms(deg_ref)
    agg = jnp.concatenate(
        [p_ref[0, 0, :N] + p_ref[0, 1, :N], p_ref[1, 0, :N] + p_ref[1, 1, :N]],
        axis=1)
    h = jnp.maximum(agg * norm_in[:, None] + b1_ref[...][None, :], 0.0)
    y_ref[...] = jnp.dot(h * norm_out[:, None], w_ref[...],
                         preferred_element_type=jnp.float32)


def _final_body(deg_ref, q_ref, b2_ref, out_ref):
    _, norm_in = _norms(deg_ref)
    agg = q_ref[0, 0, :N] + q_ref[0, 1, :N]
    out_ref[...] = agg * norm_in[:, None] + b2_ref[...][None, :]


def kernel(x, edge_index, W1, b1, W2, b2):
    E = edge_index.shape[1]
    K = -(-E // (NW * CHUNK))
    K = -(-K // (2 * _G)) * (2 * _G)
    pad = NW * K * CHUNK - E
    src = jnp.concatenate([edge_index[0], jnp.zeros((pad,), jnp.int32)])
    dst = jnp.concatenate([edge_index[1], jnp.full((pad,), N, jnp.int32)])
    src = src.reshape(NW, K, CHUNK)
    dst = dst.reshape(NW, K, CHUNK)

    deg = _make_degree_kernel(K)(src, dst)

    y1a, y1b = pl.pallas_call(
        _mm1_body,
        out_shape=[jax.ShapeDtypeStruct((N, D_OUT), jnp.float32),
                   jax.ShapeDtypeStruct((N, D_OUT), jnp.float32)],
    )(deg, x, W1)

    p = _make_edge_kernel(K, 2)(y1a, y1b, src, dst)

    y2 = pl.pallas_call(
        _mm2_body,
        out_shape=jax.ShapeDtypeStruct((N, D_OUT), jnp.float32),
    )(deg, p, b1, W2)

    q = _make_edge_kernel(K, 1)(y2, src, dst)

    out = pl.pallas_call(
        _final_body,
        out_shape=jax.ShapeDtypeStruct((N, D_OUT), jnp.float32),
    )(deg, q, b2)

    return out


# speedup vs baseline: 2.2112x; 2.2112x over previous
import functools

import jax
import jax.numpy as jnp
from jax import lax
from jax.experimental import pallas as pl
from jax.experimental.pallas import tpu as pltpu
from jax.experimental.pallas import tpu_sc as plsc

N = 10000
D_H = 128
D_OUT = 64

NC = 2
NS = 16
NW = NC * NS
CHUNK = 128
N_ACC = 10240
ZROWS = N_ACC // NS

_MESH = plsc.VectorSubcoreMesh(
    core_axis_name="c", subcore_axis_name="s", num_cores=NC, num_subcores=NS)


def _zero_f32(ref, rows, width):
    zv = jnp.zeros((16,), jnp.float32)

    @pl.loop(0, rows)
    def _row(r):
        for k in range(width // 16):
            ref[r, pl.ds(k * 16, 16)] = zv


def _make_degree_kernel(K):
    @functools.partial(
        pl.kernel,
        out_type=jax.ShapeDtypeStruct((NC, 2, N_ACC), jnp.float32),
        mesh=_MESH,
        scratch_types=[
            pltpu.VMEM((K, CHUNK), jnp.int32),
            pltpu.VMEM((1, CHUNK), jnp.float32),
            pltpu.VMEM((1, ZROWS), jnp.float32),
            pltpu.VMEM_SHARED((N_ACC,), jnp.float32),
            pltpu.VMEM_SHARED((N_ACC,), jnp.float32),
        ],
    )
    def deg_kernel(src_hbm, dst_hbm, deg_hbm, idx_v, ones_v, zbuf_v, acc_out, acc_in):
        c = lax.axis_index("c")
        s = lax.axis_index("s")
        wid = c * NS + s

        _zero_f32(zbuf_v, 1, ZROWS)
        for k in range(CHUNK // 16):
            ones_v[0, pl.ds(k * 16, 16)] = jnp.ones((16,), jnp.float32)
        pltpu.sync_copy(zbuf_v.at[0], acc_out.at[pl.ds(s * ZROWS, ZROWS)])
        pltpu.sync_copy(zbuf_v.at[0], acc_in.at[pl.ds(s * ZROWS, ZROWS)])
        plsc.subcore_barrier()

        pltpu.sync_copy(src_hbm.at[wid], idx_v)

        @pl.loop(0, K)
        def _src_chunk(j):
            pltpu.sync_copy(ones_v.at[0], acc_out.at[idx_v.at[j]], add=True)

        pltpu.sync_copy(dst_hbm.at[wid], idx_v)

        @pl.loop(0, K)
        def _dst_chunk(j):
            pltpu.sync_copy(ones_v.at[0], acc_in.at[idx_v.at[j]], add=True)

        plsc.subcore_barrier()
        pltpu.sync_copy(acc_out.at[pl.ds(s * ZROWS, ZROWS)],
                        deg_hbm.at[c, 0, pl.ds(s * ZROWS, ZROWS)])
        pltpu.sync_copy(acc_in.at[pl.ds(s * ZROWS, ZROWS)],
                        deg_hbm.at[c, 1, pl.ds(s * ZROWS, ZROWS)])

    return deg_kernel


_G = 1


def _make_edge_kernel(K, nsrc, pipelined=True):
    D = D_OUT
    assert K % (2 * _G) == 0
    NG = K // _G

    @functools.partial(
        pl.kernel,
        out_type=jax.ShapeDtypeStruct((nsrc, NC, N_ACC, D), jnp.float32),
        mesh=_MESH,
        scratch_types=[
            pltpu.VMEM((K, CHUNK), jnp.int32),
            pltpu.VMEM((K, CHUNK), jnp.int32),
            pltpu.VMEM((2 * _G, CHUNK, D), jnp.float32),
            pltpu.VMEM((128, D), jnp.float32),
            pltpu.VMEM_SHARED((N_ACC, D), jnp.float32),
            pltpu.VMEM_SHARED((N_ACC, D), jnp.float32),
            pltpu.SemaphoreType.DMA,
            pltpu.SemaphoreType.DMA,
        ],
        compiler_params=pltpu.CompilerParams(use_tc_tiling_on_sc=False),
    )
    def edge_kernel(*refs):
        ys = refs[:nsrc]
        (src_hbm, dst_hbm, out_hbm, idx_s, idx_d, gbuf, zbuf, acc, tbl,
         sg0, sg1) = refs[nsrc:]
        sg = (sg0, sg1)
        c = lax.axis_index("c")
        s = lax.axis_index("s")
        wid = c * NS + s

        _zero_f32(zbuf, 128, D)
        pltpu.sync_copy(src_hbm.at[wid], idx_s)
        pltpu.sync_copy(dst_hbm.at[wid], idx_d)

        NROWS = N // NS

        for p in range(nsrc):
            y_hbm = ys[p]

            def fire_g(st, i):
                for b in range(_G):
                    pltpu.async_copy(tbl.at[idx_s.at[i * _G + b]],
                                     gbuf.at[st * _G + b], sg[st])

            def drain_g(st, i):
                for b in range(_G):
                    pltpu.make_async_copy(tbl.at[idx_s.at[i * _G + b]],
                                          gbuf.at[st * _G + b], sg[st]).wait()

            def scatter(st, i):
                for b in range(_G):
                    pltpu.sync_copy(gbuf.at[st * _G + b],
                                    acc.at[idx_d.at[i * _G + b]], add=True)

            pltpu.sync_copy(y_hbm.at[pl.ds(s * NROWS, NROWS)],
                            tbl.at[pl.ds(s * NROWS, NROWS)])
            for b in range(ZROWS // 128):
                pltpu.sync_copy(zbuf, acc.at[pl.ds(s * ZROWS + b * 128, 128)])
            plsc.subcore_barrier()

            if not pipelined:
                @pl.loop(0, K)
                def _chunk(j):
                    pltpu.sync_copy(tbl.at[idx_s.at[j]], gbuf.at[0])
                    pltpu.sync_copy(gbuf.at[0], acc.at[idx_d.at[j]], add=True)

                plsc.subcore_barrier()
                pltpu.sync_copy(acc.at[pl.ds(s * ZROWS, ZROWS)],
                                out_hbm.at[p, c, pl.ds(s * ZROWS, ZROWS)])
                continue

            fire_g(0, 0)
            fire_g(1, 1)

            @pl.loop(0, NG, step=2)
            def _pair(i):
                drain_g(0, i)
                scatter(0, i)

                @pl.when(i + 2 < NG)
                def _():
                    fire_g(0, i + 2)

                drain_g(1, i + 1)
                scatter(1, i + 1)

                @pl.when(i + 3 < NG)
                def _():
                    fire_g(1, i + 3)

            plsc.subcore_barrier()
            pltpu.sync_copy(acc.at[pl.ds(s * ZROWS, ZROWS)],
                            out_hbm.at[p, c, pl.ds(s * ZROWS, ZROWS)])

    return edge_kernel


def _norms(deg_ref):
    deg_out = deg_ref[0, 0, :N] + deg_ref[1, 0, :N]
    deg_in = deg_ref[0, 1, :N] + deg_ref[1, 1, :N]
    norm_out = jnp.where(deg_out > 0, lax.rsqrt(jnp.maximum(deg_out, 1.0)), 0.0)
    norm_in = jnp.where(deg_in > 0, lax.rsqrt(jnp.maximum(deg_in, 1.0)), 0.0)
    return norm_out, norm_in


def _mm1_body(deg_ref, x_ref, w_ref, ya_ref, yb_ref):
    norm_out, _ = _norms(deg_ref)
    y = jnp.dot(x_ref[...] * norm_out[:, None], w_ref[...],
                preferred_element_type=jnp.float32)
    ya_ref[...] = y[:, :D_OUT]
    yb_ref[...] = y[:, D_OUT:]


def _mm2_body(deg_ref, p_ref, b1_ref, w_ref, y_ref):
    norm_out, norm_in = _norms(deg_ref)
    agg = jnp.concatenate(
        [p_ref[0, 0, :N] + p_ref[0, 1, :N], p_ref[1, 0, :N] + p_ref[1, 1, :N]],
        axis=1)
    h = jnp.maximum(agg * norm_in[:, None] + b1_ref[...][None, :], 0.0)
    y_ref[...] = jnp.dot(h * norm_out[:, None], w_ref[...],
                         preferred_element_type=jnp.float32)


def _final_body(deg_ref, q_ref, b2_ref, out_ref):
    _, norm_in = _norms(deg_ref)
    agg = q_ref[0, 0, :N] + q_ref[0, 1, :N]
    out_ref[...] = agg * norm_in[:, None] + b2_ref[...][None, :]


def kernel(x, edge_index, W1, b1, W2, b2):
    E = edge_index.shape[1]
    K = -(-E // (NW * CHUNK))
    K = -(-K // (2 * _G)) * (2 * _G)
    pad = NW * K * CHUNK - E
    src = jnp.concatenate([edge_index[0], jnp.zeros((pad,), jnp.int32)])
    dst = jnp.concatenate([edge_index[1], jnp.full((pad,), N, jnp.int32)])
    src = src.reshape(NW, K, CHUNK)
    dst = dst.reshape(NW, K, CHUNK)

    deg = _make_degree_kernel(K)(src, dst)

    y1a, y1b = pl.pallas_call(
        _mm1_body,
        out_shape=[jax.ShapeDtypeStruct((N, D_OUT), jnp.float32),
                   jax.ShapeDtypeStruct((N, D_OUT), jnp.float32)],
    )(deg, x, W1)

    p = _make_edge_kernel(K, 2)(y1a, y1b, src, dst)

    y2 = pl.pallas_call(
        _mm2_body,
        out_shape=jax.ShapeDtypeStruct((N, D_OUT), jnp.float32),
    )(deg, p, b1, W2)

    q = _make_edge_kernel(K, 1)(y2, src, dst)

    out = pl.pallas_call(
        _final_body,
        out_shape=jax.ShapeDtypeStruct((N, D_OUT), jnp.float32),
    )(deg, q, b2)

    return out


# speedup vs baseline: 2.6772x; 1.2108x over previous
import functools

import jax
import jax.numpy as jnp
from jax import lax
from jax.experimental import pallas as pl
from jax.experimental.pallas import tpu as pltpu
from jax.experimental.pallas import tpu_sc as plsc

N = 10000
D_H = 128
D_OUT = 64

NC = 2
NS = 16
NW = NC * NS
CHUNK = 128
N_ACC = 10240
ZROWS = N_ACC // NS
TROWS = N // NS

_MESH = plsc.VectorSubcoreMesh(
    core_axis_name="c", subcore_axis_name="s", num_cores=NC, num_subcores=NS)

_SC_PARAMS = pltpu.CompilerParams(use_tc_tiling_on_sc=False)


def _zero_f32(ref, rows, width):
    zv = jnp.zeros((16,), jnp.float32)

    @pl.loop(0, rows)
    def _row(r):
        for k in range(width // 16):
            ref[r, pl.ds(k * 16, 16)] = zv


def _chunk_split(C):
    B0, R = divmod(C, NW)
    KMAX = B0 + (1 if R else 0)
    return B0, R, KMAX


def _load_idx(ei_hbm, dim, dst, wid, start, B0, R, KMAX):
    if R:
        @pl.when(wid < R)
        def _():
            pltpu.sync_copy(ei_hbm.at[dim, pl.ds(start, KMAX)], dst)

        @pl.when(wid >= R)
        def _():
            pltpu.sync_copy(ei_hbm.at[dim, pl.ds(start, B0)],
                            dst.at[pl.ds(0, B0)])
    else:
        pltpu.sync_copy(ei_hbm.at[dim, pl.ds(start, B0)], dst)


def _make_degree_kernel(C):
    B0, R, KMAX = _chunk_split(C)

    @functools.partial(
        pl.kernel,
        out_type=jax.ShapeDtypeStruct((NC, 2, N_ACC), jnp.float32),
        mesh=_MESH,
        scratch_types=[
            pltpu.VMEM((KMAX, CHUNK), jnp.int32),
            pltpu.VMEM((1, CHUNK), jnp.float32),
            pltpu.VMEM((1, ZROWS), jnp.float32),
            pltpu.VMEM_SHARED((N_ACC,), jnp.float32),
            pltpu.VMEM_SHARED((N_ACC,), jnp.float32),
        ],
        compiler_params=_SC_PARAMS,
    )
    def deg_kernel(ei_hbm, deg_hbm, idx_v, ones_v, zbuf_v, acc_out, acc_in):
        c = lax.axis_index("c")
        s = lax.axis_index("s")
        wid = c * NS + s
        start = B0 * wid + jnp.minimum(wid, R)
        kc = jnp.where(wid < R, B0 + 1, B0)

        _zero_f32(zbuf_v, 1, ZROWS)
        for k in range(CHUNK // 16):
            ones_v[0, pl.ds(k * 16, 16)] = jnp.ones((16,), jnp.float32)
        pltpu.sync_copy(zbuf_v.at[0], acc_out.at[pl.ds(s * ZROWS, ZROWS)])
        pltpu.sync_copy(zbuf_v.at[0], acc_in.at[pl.ds(s * ZROWS, ZROWS)])
        plsc.subcore_barrier()

        _load_idx(ei_hbm, 0, idx_v, wid, start, B0, R, KMAX)

        @pl.loop(0, kc)
        def _src_chunk(j):
            pltpu.sync_copy(ones_v.at[0], acc_out.at[idx_v.at[j]], add=True)

        _load_idx(ei_hbm, 1, idx_v, wid, start, B0, R, KMAX)

        @pl.loop(0, kc)
        def _dst_chunk(j):
            pltpu.sync_copy(ones_v.at[0], acc_in.at[idx_v.at[j]], add=True)

        plsc.subcore_barrier()
        pltpu.sync_copy(acc_out.at[pl.ds(s * ZROWS, ZROWS)],
                        deg_hbm.at[c, 0, pl.ds(s * ZROWS, ZROWS)])
        pltpu.sync_copy(acc_in.at[pl.ds(s * ZROWS, ZROWS)],
                        deg_hbm.at[c, 1, pl.ds(s * ZROWS, ZROWS)])

    return deg_kernel


def _make_edge_kernel(C, nsrc):
    D = D_OUT
    B0, R, KMAX = _chunk_split(C)

    @functools.partial(
        pl.kernel,
        out_type=jax.ShapeDtypeStruct((nsrc, NC, N_ACC, D), jnp.float32),
        mesh=_MESH,
        scratch_types=[
            pltpu.VMEM((KMAX, CHUNK), jnp.int32),
            pltpu.VMEM((KMAX, CHUNK), jnp.int32),
            pltpu.VMEM((3, CHUNK, D), jnp.float32),
            pltpu.VMEM_SHARED((N_ACC, D), jnp.float32),
            pltpu.VMEM_SHARED((N_ACC, D), jnp.float32),
            pltpu.SemaphoreType.DMA,
            pltpu.SemaphoreType.DMA,
            pltpu.SemaphoreType.DMA,
            pltpu.SemaphoreType.DMA,
            pltpu.SemaphoreType.DMA,
            pltpu.SemaphoreType.DMA,
        ],
        compiler_params=_SC_PARAMS,
    )
    def edge_kernel(*refs):
        ys = refs[:nsrc]
        (ei_hbm, out_hbm, idx_s, idx_d, gbuf, acc, tbl,
         sg0, sg1, sg2, ss0, ss1, ss2) = refs[nsrc:]
        sg = (sg0, sg1, sg2)
        ss = (ss0, ss1, ss2)
        c = lax.axis_index("c")
        s = lax.axis_index("s")
        wid = c * NS + s
        start = B0 * wid + jnp.minimum(wid, R)
        kc = jnp.where(wid < R, B0 + 1, B0)

        _load_idx(ei_hbm, 0, idx_s, wid, start, B0, R, KMAX)
        _load_idx(ei_hbm, 1, idx_d, wid, start, B0, R, KMAX)

        def fire_g(b, m):
            pltpu.async_copy(tbl.at[idx_s.at[m]], gbuf.at[b], sg[b])

        def drain_g(b):
            pltpu.make_async_copy(tbl.at[idx_s.at[0]], gbuf.at[b], sg[b]).wait()

        def fire_s(b, m):
            pltpu.async_copy(gbuf.at[b], acc.at[idx_d.at[m]], ss[b], add=True)

        def drain_s(b):
            pltpu.make_async_copy(gbuf.at[b], acc.at[idx_d.at[0]], ss[b]).wait()

        for p in range(nsrc):
            y_hbm = ys[p]

            _zero_f32(gbuf.at[0], 128, D)
            pltpu.sync_copy(y_hbm.at[pl.ds(s * TROWS, TROWS)],
                            tbl.at[pl.ds(s * TROWS, TROWS)])
            for b in range(ZROWS // 128):
                pltpu.sync_copy(gbuf.at[0], acc.at[pl.ds(s * ZROWS + b * 128, 128)])
            plsc.subcore_barrier()

            fire_g(0, 0)
            fire_g(1, 1)

            @pl.loop(0, kc, step=3)
            def _trip(j):
                for off in range(3):
                    m = j + off
                    nb = (off + 2) % 3

                    @pl.when(m < kc)
                    def _():
                        drain_g(off)
                        fire_s(off, m)

                        @pl.when(m + 2 < kc)
                        def _():
                            @pl.when(m - 1 >= 0)
                            def _():
                                drain_s(nb)

                            fire_g(nb, m + 2)

            for off in range(3):
                @pl.when(off < kc)
                def _():
                    drain_s(off)

            plsc.subcore_barrier()
            pltpu.sync_copy(acc.at[pl.ds(s * ZROWS, ZROWS)],
                            out_hbm.at[p, c, pl.ds(s * ZROWS, ZROWS)])

    return edge_kernel


def _norms(deg_ref):
    deg_out = deg_ref[0, 0, :N] + deg_ref[1, 0, :N]
    deg_in = deg_ref[0, 1, :N] + deg_ref[1, 1, :N]
    norm_out = jnp.where(deg_out > 0, lax.rsqrt(jnp.maximum(deg_out, 1.0)), 0.0)
    norm_in = jnp.where(deg_in > 0, lax.rsqrt(jnp.maximum(deg_in, 1.0)), 0.0)
    return norm_out, norm_in


def _mm1_body(deg_ref, x_ref, w_ref, ya_ref, yb_ref):
    norm_out, _ = _norms(deg_ref)
    y = jnp.dot(x_ref[...] * norm_out[:, None], w_ref[...],
                preferred_element_type=jnp.float32)
    ya_ref[...] = y[:, :D_OUT]
    yb_ref[...] = y[:, D_OUT:]


def _mm2_body(deg_ref, p_ref, b1_ref, w_ref, y_ref):
    norm_out, norm_in = _norms(deg_ref)
    agg = jnp.concatenate(
        [p_ref[0, 0, :N] + p_ref[0, 1, :N], p_ref[1, 0, :N] + p_ref[1, 1, :N]],
        axis=1)
    h = jnp.maximum(agg * norm_in[:, None] + b1_ref[...][None, :], 0.0)
    y_ref[...] = jnp.dot(h * norm_out[:, None], w_ref[...],
                         preferred_element_type=jnp.float32)


def _final_body(deg_ref, q_ref, b2_ref, out_ref):
    _, norm_in = _norms(deg_ref)
    agg = q_ref[0, 0, :N] + q_ref[0, 1, :N]
    out_ref[...] = agg * norm_in[:, None] + b2_ref[...][None, :]


def kernel(x, edge_index, W1, b1, W2, b2):
    E = edge_index.shape[1]
    assert E % CHUNK == 0
    C = E // CHUNK
    ei3 = edge_index.reshape(2, C, CHUNK)

    deg = _make_degree_kernel(C)(ei3)

    y1a, y1b = pl.pallas_call(
        _mm1_body,
        out_shape=[jax.ShapeDtypeStruct((N, D_OUT), jnp.float32),
                   jax.ShapeDtypeStruct((N, D_OUT), jnp.float32)],
    )(deg, x, W1)

    p = _make_edge_kernel(C, 2)(y1a, y1b, ei3)

    y2 = pl.pallas_call(
        _mm2_body,
        out_shape=jax.ShapeDtypeStruct((N, D_OUT), jnp.float32),
    )(deg, p, b1, W2)

    q = _make_edge_kernel(C, 1)(y2, ei3)

    out = pl.pallas_call(
        _final_body,
        out_shape=jax.ShapeDtypeStruct((N, D_OUT), jnp.float32),
    )(deg, q, b2)

    return out


# speedup vs baseline: 2.7617x; 1.0315x over previous
import functools

import jax
import jax.numpy as jnp
from jax import lax
from jax.experimental import pallas as pl
from jax.experimental.pallas import tpu as pltpu
from jax.experimental.pallas import tpu_sc as plsc

N = 10000
D_H = 128
D_OUT = 64

NC = 2
NS = 16
NW = NC * NS
CHUNK = 128
N_ACC = 10240
ZROWS = N_ACC // NS
TROWS = N // NS

_MESH = plsc.VectorSubcoreMesh(
    core_axis_name="c", subcore_axis_name="s", num_cores=NC, num_subcores=NS)

_SC_PARAMS = pltpu.CompilerParams(use_tc_tiling_on_sc=False)


def _zero_f32(ref, rows, width):
    zv = jnp.zeros((16,), jnp.float32)

    @pl.loop(0, rows)
    def _row(r):
        for k in range(width // 16):
            ref[r, pl.ds(k * 16, 16)] = zv


def _chunk_split(C):
    B0, R = divmod(C, NW)
    KMAX = B0 + (1 if R else 0)
    return B0, R, KMAX


def _load_idx(ei_hbm, dim, dst, wid, start, B0, R, KMAX):
    if R:
        @pl.when(wid < R)
        def _():
            pltpu.sync_copy(ei_hbm.at[dim, pl.ds(start, KMAX)], dst)

        @pl.when(wid >= R)
        def _():
            pltpu.sync_copy(ei_hbm.at[dim, pl.ds(start, B0)],
                            dst.at[pl.ds(0, B0)])
    else:
        pltpu.sync_copy(ei_hbm.at[dim, pl.ds(start, B0)], dst)


def _make_degree_kernel(C):
    B0, R, KMAX = _chunk_split(C)

    @functools.partial(
        pl.kernel,
        out_type=jax.ShapeDtypeStruct((NC, 2, N_ACC), jnp.float32),
        mesh=_MESH,
        scratch_types=[
            pltpu.VMEM((KMAX, CHUNK), jnp.int32),
            pltpu.VMEM((KMAX, CHUNK), jnp.int32),
            pltpu.VMEM((1, CHUNK), jnp.float32),
            pltpu.VMEM((1, ZROWS), jnp.float32),
            pltpu.VMEM_SHARED((N_ACC,), jnp.float32),
            pltpu.VMEM_SHARED((N_ACC,), jnp.float32),
            pltpu.SemaphoreType.DMA,
            pltpu.SemaphoreType.DMA,
        ],
        compiler_params=_SC_PARAMS,
    )
    def deg_kernel(ei_hbm, deg_hbm, idx_s, idx_d, ones_v, zbuf_v,
                   acc_out, acc_in, sa, sb):
        c = lax.axis_index("c")
        s = lax.axis_index("s")
        wid = c * NS + s
        start = B0 * wid + jnp.minimum(wid, R)
        kc = jnp.where(wid < R, B0 + 1, B0)

        _zero_f32(zbuf_v, 1, ZROWS)
        for k in range(CHUNK // 16):
            ones_v[0, pl.ds(k * 16, 16)] = jnp.ones((16,), jnp.float32)
        pltpu.sync_copy(zbuf_v.at[0], acc_out.at[pl.ds(s * ZROWS, ZROWS)])
        pltpu.sync_copy(zbuf_v.at[0], acc_in.at[pl.ds(s * ZROWS, ZROWS)])
        plsc.subcore_barrier()

        _load_idx(ei_hbm, 0, idx_s, wid, start, B0, R, KMAX)
        _load_idx(ei_hbm, 1, idx_d, wid, start, B0, R, KMAX)

        GRP = 8

        @pl.loop(0, kc, step=GRP)
        def _grp(j):
            for t in range(GRP):
                @pl.when(j + t < kc)
                def _():
                    pltpu.async_copy(ones_v.at[0], acc_out.at[idx_s.at[j + t]],
                                     sa, add=True)
                    pltpu.async_copy(ones_v.at[0], acc_in.at[idx_d.at[j + t]],
                                     sb, add=True)
            for t in range(GRP):
                @pl.when(j + t < kc)
                def _():
                    pltpu.make_async_copy(
                        ones_v.at[0], acc_out.at[idx_s.at[0]], sa).wait()
                    pltpu.make_async_copy(
                        ones_v.at[0], acc_in.at[idx_d.at[0]], sb).wait()

        plsc.subcore_barrier()
        pltpu.sync_copy(acc_out.at[pl.ds(s * ZROWS, ZROWS)],
                        deg_hbm.at[c, 0, pl.ds(s * ZROWS, ZROWS)])
        pltpu.sync_copy(acc_in.at[pl.ds(s * ZROWS, ZROWS)],
                        deg_hbm.at[c, 1, pl.ds(s * ZROWS, ZROWS)])

    return deg_kernel


def _make_edge_kernel(C, nsrc):
    D = D_OUT
    B0, R, KMAX = _chunk_split(C)

    @functools.partial(
        pl.kernel,
        out_type=jax.ShapeDtypeStruct((nsrc, NC, N_ACC, D), jnp.float32),
        mesh=_MESH,
        scratch_types=[
            pltpu.VMEM((KMAX, CHUNK), jnp.int32),
            pltpu.VMEM((KMAX, CHUNK), jnp.int32),
            pltpu.VMEM((3, CHUNK, D), jnp.float32),
            pltpu.VMEM_SHARED((N_ACC, D), jnp.float32),
            pltpu.VMEM_SHARED((N_ACC, D), jnp.float32),
            pltpu.SemaphoreType.DMA,
            pltpu.SemaphoreType.DMA,
            pltpu.SemaphoreType.DMA,
            pltpu.SemaphoreType.DMA,
            pltpu.SemaphoreType.DMA,
            pltpu.SemaphoreType.DMA,
        ],
        compiler_params=_SC_PARAMS,
    )
    def edge_kernel(*refs):
        ys = refs[:nsrc]
        (ei_hbm, out_hbm, idx_s, idx_d, gbuf, acc, tbl,
         sg0, sg1, sg2, ss0, ss1, ss2) = refs[nsrc:]
        sg = (sg0, sg1, sg2)
        ss = (ss0, ss1, ss2)
        c = lax.axis_index("c")
        s = lax.axis_index("s")
        wid = c * NS + s
        start = B0 * wid + jnp.minimum(wid, R)
        kc = jnp.where(wid < R, B0 + 1, B0)

        _load_idx(ei_hbm, 0, idx_s, wid, start, B0, R, KMAX)
        _load_idx(ei_hbm, 1, idx_d, wid, start, B0, R, KMAX)

        def fire_g(b, m):
            pltpu.async_copy(tbl.at[idx_s.at[m]], gbuf.at[b], sg[b])

        def drain_g(b):
            pltpu.make_async_copy(tbl.at[idx_s.at[0]], gbuf.at[b], sg[b]).wait()

        def fire_s(b, m):
            pltpu.async_copy(gbuf.at[b], acc.at[idx_d.at[m]], ss[b], add=True)

        def drain_s(b):
            pltpu.make_async_copy(gbuf.at[b], acc.at[idx_d.at[0]], ss[b]).wait()

        for p in range(nsrc):
            y_hbm = ys[p]

            _zero_f32(gbuf.at[0], 128, D)
            pltpu.sync_copy(y_hbm.at[pl.ds(s * TROWS, TROWS)],
                            tbl.at[pl.ds(s * TROWS, TROWS)])
            for b in range(ZROWS // 128):
                pltpu.sync_copy(gbuf.at[0], acc.at[pl.ds(s * ZROWS + b * 128, 128)])
            plsc.subcore_barrier()

            fire_g(0, 0)
            fire_g(1, 1)

            @pl.loop(0, kc, step=3)
            def _trip(j):
                for off in range(3):
                    m = j + off
                    nb = (off + 2) % 3

                    @pl.when(m < kc)
                    def _():
                        drain_g(off)
                        fire_s(off, m)

                        @pl.when(m + 2 < kc)
                        def _():
                            @pl.when(m - 1 >= 0)
                            def _():
                                drain_s(nb)

                            fire_g(nb, m + 2)

            for off in range(3):
                @pl.when(off < kc)
                def _():
                    drain_s(off)

            plsc.subcore_barrier()
            pltpu.sync_copy(acc.at[pl.ds(s * ZROWS, ZROWS)],
                            out_hbm.at[p, c, pl.ds(s * ZROWS, ZROWS)])

    return edge_kernel


def _norms(deg_ref):
    deg_out = deg_ref[0, 0, :N] + deg_ref[1, 0, :N]
    deg_in = deg_ref[0, 1, :N] + deg_ref[1, 1, :N]
    norm_out = jnp.where(deg_out > 0, lax.rsqrt(jnp.maximum(deg_out, 1.0)), 0.0)
    norm_in = jnp.where(deg_in > 0, lax.rsqrt(jnp.maximum(deg_in, 1.0)), 0.0)
    return norm_out, norm_in


def _mm1_body(deg_ref, x_ref, w_ref, ya_ref, yb_ref):
    norm_out, _ = _norms(deg_ref)
    y = jnp.dot(x_ref[...] * norm_out[:, None], w_ref[...],
                preferred_element_type=jnp.float32)
    ya_ref[...] = y[:, :D_OUT]
    yb_ref[...] = y[:, D_OUT:]


def _mm2_body(deg_ref, p_ref, b1_ref, w_ref, y_ref):
    norm_out, norm_in = _norms(deg_ref)
    agg = jnp.concatenate(
        [p_ref[0, 0, :N] + p_ref[0, 1, :N], p_ref[1, 0, :N] + p_ref[1, 1, :N]],
        axis=1)
    h = jnp.maximum(agg * norm_in[:, None] + b1_ref[...][None, :], 0.0)
    y_ref[...] = jnp.dot(h * norm_out[:, None], w_ref[...],
                         preferred_element_type=jnp.float32)


def _final_body(deg_ref, q_ref, b2_ref, out_ref):
    _, norm_in = _norms(deg_ref)
    agg = q_ref[0, 0, :N] + q_ref[0, 1, :N]
    out_ref[...] = agg * norm_in[:, None] + b2_ref[...][None, :]


def kernel(x, edge_index, W1, b1, W2, b2):
    E = edge_index.shape[1]
    assert E % CHUNK == 0
    C = E // CHUNK
    ei3 = edge_index.reshape(2, C, CHUNK)

    deg = _make_degree_kernel(C)(ei3)

    y1a, y1b = pl.pallas_call(
        _mm1_body,
        out_shape=[jax.ShapeDtypeStruct((N, D_OUT), jnp.float32),
                   jax.ShapeDtypeStruct((N, D_OUT), jnp.float32)],
    )(deg, x, W1)

    p = _make_edge_kernel(C, 2)(y1a, y1b, ei3)

    y2 = pl.pallas_call(
        _mm2_body,
        out_shape=jax.ShapeDtypeStruct((N, D_OUT), jnp.float32),
    )(deg, p, b1, W2)

    q = _make_edge_kernel(C, 1)(y2, ei3)

    out = pl.pallas_call(
        _final_body,
        out_shape=jax.ShapeDtypeStruct((N, D_OUT), jnp.float32),
    )(deg, q, b2)

    return out


# speedup vs baseline: 3.0797x; 1.1152x over previous
import functools

import jax
import jax.numpy as jnp
from jax import lax
from jax.experimental import pallas as pl
from jax.experimental.pallas import tpu as pltpu
from jax.experimental.pallas import tpu_sc as plsc

N = 10000
D_H = 128
D_OUT = 64

NC = 2
NS = 16
NW = NC * NS
CHUNK = 128
N_ACC = 10240
ZROWS = N_ACC // NS
TROWS = N // NS

_MESH = plsc.VectorSubcoreMesh(
    core_axis_name="c", subcore_axis_name="s", num_cores=NC, num_subcores=NS)

_SC_PARAMS = pltpu.CompilerParams(use_tc_tiling_on_sc=False)


def _zero_f32(ref, rows, width):
    zv = jnp.zeros((16,), jnp.float32)

    @pl.loop(0, rows)
    def _row(r):
        for k in range(width // 16):
            ref[r, pl.ds(k * 16, 16)] = zv


def _chunk_split(C):
    B0, R = divmod(C, NW)
    KMAX = B0 + (1 if R else 0)
    return B0, R, KMAX


def _load_idx(ei_hbm, dim, dst, wid, start, B0, R, KMAX):
    if R:
        @pl.when(wid < R)
        def _():
            pltpu.sync_copy(ei_hbm.at[dim, pl.ds(start, KMAX)], dst)

        @pl.when(wid >= R)
        def _():
            pltpu.sync_copy(ei_hbm.at[dim, pl.ds(start, B0)],
                            dst.at[pl.ds(0, B0)])
    else:
        pltpu.sync_copy(ei_hbm.at[dim, pl.ds(start, B0)], dst)


def _make_degree_kernel(C):
    B0, R, KMAX = _chunk_split(C)

    @functools.partial(
        pl.kernel,
        out_type=jax.ShapeDtypeStruct((NC, 2, N_ACC), jnp.float32),
        mesh=_MESH,
        scratch_types=[
            pltpu.VMEM((KMAX, CHUNK), jnp.int32),
            pltpu.VMEM((KMAX, CHUNK), jnp.int32),
            pltpu.VMEM((1, CHUNK), jnp.float32),
            pltpu.VMEM((1, ZROWS), jnp.float32),
            pltpu.VMEM_SHARED((N_ACC,), jnp.float32),
            pltpu.VMEM_SHARED((N_ACC,), jnp.float32),
            pltpu.SemaphoreType.DMA,
            pltpu.SemaphoreType.DMA,
        ],
        compiler_params=_SC_PARAMS,
    )
    def deg_kernel(ei_hbm, deg_hbm, idx_s, idx_d, ones_v, zbuf_v,
                   acc_out, acc_in, sa, sb):
        c = lax.axis_index("c")
        s = lax.axis_index("s")
        wid = c * NS + s
        start = B0 * wid + jnp.minimum(wid, R)
        kc = jnp.where(wid < R, B0 + 1, B0)

        _zero_f32(zbuf_v, 1, ZROWS)
        for k in range(CHUNK // 16):
            ones_v[0, pl.ds(k * 16, 16)] = jnp.ones((16,), jnp.float32)
        pltpu.sync_copy(zbuf_v.at[0], acc_out.at[pl.ds(s * ZROWS, ZROWS)])
        pltpu.sync_copy(zbuf_v.at[0], acc_in.at[pl.ds(s * ZROWS, ZROWS)])
        plsc.subcore_barrier()

        _load_idx(ei_hbm, 0, idx_s, wid, start, B0, R, KMAX)
        _load_idx(ei_hbm, 1, idx_d, wid, start, B0, R, KMAX)

        GRP = 8

        @pl.loop(0, kc, step=GRP)
        def _grp(j):
            for t in range(GRP):
                @pl.when(j + t < kc)
                def _():
                    pltpu.async_copy(ones_v.at[0], acc_out.at[idx_s.at[j + t]],
                                     sa, add=True)
                    pltpu.async_copy(ones_v.at[0], acc_in.at[idx_d.at[j + t]],
                                     sb, add=True)
            for t in range(GRP):
                @pl.when(j + t < kc)
                def _():
                    pltpu.make_async_copy(
                        ones_v.at[0], acc_out.at[idx_s.at[0]], sa).wait()
                    pltpu.make_async_copy(
                        ones_v.at[0], acc_in.at[idx_d.at[0]], sb).wait()

        plsc.subcore_barrier()
        pltpu.sync_copy(acc_out.at[pl.ds(s * ZROWS, ZROWS)],
                        deg_hbm.at[c, 0, pl.ds(s * ZROWS, ZROWS)])
        pltpu.sync_copy(acc_in.at[pl.ds(s * ZROWS, ZROWS)],
                        deg_hbm.at[c, 1, pl.ds(s * ZROWS, ZROWS)])

    return deg_kernel


def _make_edge_kernel(C, npass, DY):
    D = D_OUT
    npass_check = DY // D
    assert npass == npass_check
    B0, R, KMAX = _chunk_split(C)

    @functools.partial(
        pl.kernel,
        out_type=jax.ShapeDtypeStruct((NC, N_ACC, DY), jnp.float32),
        mesh=_MESH,
        scratch_types=[
            pltpu.VMEM((KMAX, CHUNK), jnp.int32),
            pltpu.VMEM((KMAX, CHUNK), jnp.int32),
            pltpu.VMEM((3, CHUNK, D), jnp.float32),
            pltpu.VMEM_SHARED((N_ACC, D), jnp.float32),
            pltpu.VMEM_SHARED((N_ACC, D), jnp.float32),
            pltpu.SemaphoreType.DMA,
            pltpu.SemaphoreType.DMA,
            pltpu.SemaphoreType.DMA,
            pltpu.SemaphoreType.DMA,
            pltpu.SemaphoreType.DMA,
            pltpu.SemaphoreType.DMA,
        ],
        compiler_params=_SC_PARAMS,
    )
    def edge_kernel(y_hbm, ei_hbm, out_hbm, idx_s, idx_d, gbuf, acc, tbl,
                    sg0, sg1, sg2, ss0, ss1, ss2):
        sg = (sg0, sg1, sg2)
        ss = (ss0, ss1, ss2)
        c = lax.axis_index("c")
        s = lax.axis_index("s")
        wid = c * NS + s
        start = B0 * wid + jnp.minimum(wid, R)
        kc = jnp.where(wid < R, B0 + 1, B0)

        _load_idx(ei_hbm, 0, idx_s, wid, start, B0, R, KMAX)
        _load_idx(ei_hbm, 1, idx_d, wid, start, B0, R, KMAX)

        def fire_g(b, m):
            pltpu.async_copy(tbl.at[idx_s.at[m]], gbuf.at[b], sg[b])

        def drain_g(b):
            pltpu.make_async_copy(tbl.at[idx_s.at[0]], gbuf.at[b], sg[b]).wait()

        def fire_s(b, m):
            pltpu.async_copy(gbuf.at[b], acc.at[idx_d.at[m]], ss[b], add=True)

        def drain_s(b):
            pltpu.make_async_copy(gbuf.at[b], acc.at[idx_d.at[0]], ss[b]).wait()

        for p in range(npass):
            _zero_f32(gbuf.at[0], 128, D)
            if npass == 1:
                pltpu.sync_copy(y_hbm.at[pl.ds(s * TROWS, TROWS)],
                                tbl.at[pl.ds(s * TROWS, TROWS)])
            else:
                pltpu.sync_copy(
                    y_hbm.at[pl.ds(s * TROWS, TROWS), pl.ds(p * D, D)],
                    tbl.at[pl.ds(s * TROWS, TROWS)])
            for b in range(ZROWS // 128):
                pltpu.sync_copy(gbuf.at[0], acc.at[pl.ds(s * ZROWS + b * 128, 128)])
            plsc.subcore_barrier()

            fire_g(0, 0)
            fire_g(1, 1)

            @pl.loop(0, kc, step=3)
            def _trip(j):
                for off in range(3):
                    m = j + off
                    nb = (off + 2) % 3

                    @pl.when(m < kc)
                    def _():
                        drain_g(off)
                        fire_s(off, m)

                        @pl.when(m + 2 < kc)
                        def _():
                            @pl.when(m - 1 >= 0)
                            def _():
                                drain_s(nb)

                            fire_g(nb, m + 2)

            for off in range(3):
                @pl.when(off < kc)
                def _():
                    drain_s(off)

            plsc.subcore_barrier()
            if npass == 1:
                pltpu.sync_copy(acc.at[pl.ds(s * ZROWS, ZROWS)],
                                out_hbm.at[c, pl.ds(s * ZROWS, ZROWS)])
            else:
                pltpu.sync_copy(
                    acc.at[pl.ds(s * ZROWS, ZROWS)],
                    out_hbm.at[c, pl.ds(s * ZROWS, ZROWS), pl.ds(p * D, D)])

    return edge_kernel


def _norms(deg_ref):
    deg_out = deg_ref[0, 0, :N] + deg_ref[1, 0, :N]
    deg_in = deg_ref[0, 1, :N] + deg_ref[1, 1, :N]
    norm_out = jnp.where(deg_out > 0, lax.rsqrt(jnp.maximum(deg_out, 1.0)), 0.0)
    norm_in = jnp.where(deg_in > 0, lax.rsqrt(jnp.maximum(deg_in, 1.0)), 0.0)
    return norm_out, norm_in


def _mm1_body(deg_ref, x_ref, w_ref, y_ref):
    norm_out, _ = _norms(deg_ref)
    y_ref[...] = jnp.dot(x_ref[...] * norm_out[:, None], w_ref[...],
                         preferred_element_type=jnp.float32)


def _mm2_body(deg_ref, p_ref, b1_ref, w_ref, y_ref):
    norm_out, norm_in = _norms(deg_ref)
    agg = p_ref[0, :N] + p_ref[1, :N]
    h = jnp.maximum(agg * norm_in[:, None] + b1_ref[...][None, :], 0.0)
    y_ref[...] = jnp.dot(h * norm_out[:, None], w_ref[...],
                         preferred_element_type=jnp.float32)


def _final_body(deg_ref, q_ref, b2_ref, out_ref):
    _, norm_in = _norms(deg_ref)
    agg = q_ref[0, :N] + q_ref[1, :N]
    out_ref[...] = agg * norm_in[:, None] + b2_ref[...][None, :]


def kernel(x, edge_index, W1, b1, W2, b2):
    E = edge_index.shape[1]
    assert E % CHUNK == 0
    C = E // CHUNK
    ei3 = edge_index.reshape(2, C, CHUNK)

    deg = _make_degree_kernel(C)(ei3)

    y1 = pl.pallas_call(
        _mm1_body,
        out_shape=jax.ShapeDtypeStruct((N, D_H), jnp.float32),
    )(deg, x, W1)

    p = _make_edge_kernel(C, 2, D_H)(y1, ei3)

    y2 = pl.pallas_call(
        _mm2_body,
        out_shape=jax.ShapeDtypeStruct((N, D_OUT), jnp.float32),
    )(deg, p, b1, W2)

    q = _make_edge_kernel(C, 1, D_OUT)(y2, ei3)

    out = pl.pallas_call(
        _final_body,
        out_shape=jax.ShapeDtypeStruct((N, D_OUT), jnp.float32),
    )(deg, q, b2)

    return out
